# baseline (speedup 1.0000x reference)
import jax
import jax.numpy as jnp
from jax.experimental import pallas as pl
from jax.experimental.pallas import tpu as pltpu

_B, _DK, _DV = 2048, 256, 256
_BB = 32
_N = _B // _BB


def _body(state_ref, q_ref, k_ref, dout_ref, y_ref, dm_ref,
          y_s, k_s, do_s):
    p = pl.program_id(0)
    j = pl.program_id(1)
    rows = pl.ds(pl.multiple_of(j * _BB, _BB), _BB)

    @pl.when(p == 0)
    def _():
        s = state_ref[...]
        q = q_ref[...]
        y_s[rows, :] = jnp.sum(s * q[:, None, :], axis=-1)
        k_s[rows, :] = k_ref[...]
        do_s[rows, :] = dout_ref[...]

    @pl.when(p == 1)
    def _():
        dm_ref[...] = do_s[rows, :][:, :, None] * k_s[rows, :][:, None, :]
        y_ref[...] = y_s[rows, :]


def kernel(state, query, key, d_out, *, interpret=False):
    def _read3(p, j):
        return (j * (1 - p) + (_N - 1) * p, 0, 0)

    def _read2(p, j):
        return (j * (1 - p) + (_N - 1) * p, 0)

    def _write2(p, j):
        return (j * p, 0)

    def _write3(p, j):
        return (j * p, 0, 0)

    y, dm = pl.pallas_call(
        _body,
        grid=(2, _N),
        in_specs=[
            pl.BlockSpec((_BB, _DV, _DK), _read3),
            pl.BlockSpec((_BB, _DK), _read2),
            pl.BlockSpec((_BB, _DK), _read2),
            pl.BlockSpec((_BB, _DV), _read2),
        ],
        out_specs=[
            pl.BlockSpec((_BB, _DV), _write2),
            pl.BlockSpec((_BB, _DV, _DK), _write3),
        ],
        out_shape=[
            jax.ShapeDtypeStruct((_B, _DV), jnp.float32),
            jax.ShapeDtypeStruct((_B, _DV, _DK), jnp.float32),
        ],
        scratch_shapes=[
            pltpu.VMEM((_B, _DV), jnp.float32),
            pltpu.VMEM((_B, _DK), jnp.float32),
            pltpu.VMEM((_B, _DV), jnp.float32),
        ],
        compiler_params=pltpu.CompilerParams(
            dimension_semantics=("arbitrary", "arbitrary"),
            vmem_limit_bytes=48 * 1024 * 1024,
        ),
        name="matrix_memory",
        interpret=interpret,
    )(state, query, key, d_out)
    return (y, dm)


# speedup vs baseline: 1.0017x; 1.0017x over previous
import jax
import jax.numpy as jnp
from jax.experimental import pallas as pl
from jax.experimental.pallas import tpu as pltpu

_B, _DK, _DV = 2048, 256, 256
_BB = 32
_N = _B // _BB


def _body(state_ref, q_ref, k_ref, dout_ref, y_ref, dm_ref):
    p = pl.program_id(0)

    @pl.when(p == 0)
    def _():
        s = state_ref[...]
        q = q_ref[...]
        y_ref[...] = jnp.sum(s * q[:, None, :], axis=-1)

    @pl.when(p == 1)
    def _():
        dm_ref[...] = dout_ref[...][:, :, None] * k_ref[...][:, None, :]


def kernel(state, query, key, d_out, *, interpret=False):
    def _read3(p, j):
        return (j * (1 - p) + (_N - 1) * p, 0, 0)

    def _read2(p, j):
        return (j * (1 - p) + (_N - 1) * p, 0)

    def _write2(p, j):
        return (j * p, 0)

    def _write3(p, j):
        return (j * p, 0, 0)

    y, dm = pl.pallas_call(
        _body,
        grid=(2, _N),
        in_specs=[
            pl.BlockSpec((_BB, _DV, _DK), _read3),
            pl.BlockSpec((_BB, _DK), _read2),
            pl.BlockSpec((_BB, _DK), _write2),
            pl.BlockSpec((_BB, _DV), _write2),
        ],
        out_specs=[
            pl.BlockSpec((_BB, _DV), _read2),
            pl.BlockSpec((_BB, _DV, _DK), _write3),
        ],
        out_shape=[
            jax.ShapeDtypeStruct((_B, _DV), jnp.float32),
            jax.ShapeDtypeStruct((_B, _DV, _DK), jnp.float32),
        ],
        compiler_params=pltpu.CompilerParams(
            dimension_semantics=("arbitrary", "arbitrary"),
            vmem_limit_bytes=48 * 1024 * 1024,
        ),
        name="matrix_memory",
        interpret=interpret,
    )(state, query, key, d_out)
    return (y, dm)
